# baseline (speedup 1.0000x reference)
import functools

import jax
import jax.numpy as jnp
from jax import lax
from jax.experimental import pallas as pl
from jax.experimental.pallas import tpu as pltpu
from jax.experimental.pallas import tpu_sc as plsc

D = 128
K = 10
NSC = 32
GB = 128


def _prep_body(ls_ref, q_ref, wq_ref, wk_ref, wv_ref, qp_ref, qk_ref, qv_ref):
    q = q_ref[...]
    rn = jax.lax.rsqrt(jnp.sum(q * q, axis=1, keepdims=True))
    qn = q * rn
    qp = jax.lax.dot_general(qn, wq_ref[...], (((1,), (1,)), ((), ())),
                             preferred_element_type=jnp.float32)
    scale = jnp.exp(ls_ref[0])
    qp_ref[...] = qp
    qk_ref[...] = scale * jax.lax.dot_general(qp, wk_ref[...], (((1,), (0,)), ((), ())),
                                              preferred_element_type=jnp.float32)
    qv_ref[...] = jax.lax.dot_general(qp, wv_ref[...], (((1,), (0,)), ((), ())),
                                      preferred_element_type=jnp.float32)


def _prep(q, wq, wk, wv, ls):
    B = q.shape[0]
    Bt = min(512, B)
    qp, qk, qv = pl.pallas_call(
        _prep_body,
        grid=(B // Bt,),
        in_specs=[
            pl.BlockSpec(memory_space=pltpu.SMEM),
            pl.BlockSpec((Bt, D), lambda b: (b, 0)),
            pl.BlockSpec((D, D), lambda b: (0, 0)),
            pl.BlockSpec((D, D), lambda b: (0, 0)),
            pl.BlockSpec((D, D), lambda b: (0, 0)),
        ],
        out_specs=[pl.BlockSpec((Bt, D), lambda b: (b, 0))] * 3,
        out_shape=[jax.ShapeDtypeStruct((B, D), jnp.float32)] * 3,
    )(ls, q, wq, wk, wv)
    return qp, qk, qv


def _vals_body(ent_ref, wv_ref, out_ref):
    ent = ent_ref[...]
    rn = jax.lax.rsqrt(jnp.sum(ent * ent, axis=1, keepdims=True))
    out_ref[...] = jax.lax.dot_general(ent * rn, wv_ref[...],
                                       (((1,), (1,)), ((), ())),
                                       preferred_element_type=jnp.float32)


def _vals_table(ent, wv, C=2048):
    n = ent.shape[0]
    return pl.pallas_call(
        _vals_body,
        grid=(pl.cdiv(n, C),),
        in_specs=[
            pl.BlockSpec((C, D), lambda c: (c, 0)),
            pl.BlockSpec((D, D), lambda c: (0, 0)),
        ],
        out_specs=pl.BlockSpec((C, D), lambda c: (c, 0)),
        out_shape=jax.ShapeDtypeStruct((n, D), jnp.float32),
    )(ent, wv)


def _insert3(lo, pr, c, C, Bt, r_refs, p_refs, partner_groups=None):
    r1_ref, r2_ref, r3_ref = r_refs
    w1_ref, w2_ref, w3_ref = p_refs
    RB = 32
    for i in range(Bt // RB):
        sl = pl.ds(i * RB, RB)
        r1 = r1_ref[sl, :]
        r2 = r2_ref[sl, :]
        r3 = r3_ref[sl, :]
        w1 = w1_ref[sl, :]
        w2 = w2_ref[sl, :]
        w3 = w3_ref[sl, :]
        for g in range(C // 128):
            x = lo[i * RB:(i + 1) * RB, g * 128:(g + 1) * 128]
            if pr is not None:
                wx = pr[i * RB:(i + 1) * RB, g * 128:(g + 1) * 128]
            else:
                wx = partner_groups(g)
            m = x > r1
            d = jnp.where(m, r1, x)
            dw = jnp.where(m, w1, wx)
            r1 = jnp.where(m, x, r1)
            w1 = jnp.where(m, wx, w1)
            m2 = d > r2
            d2 = jnp.where(m2, r2, d)
            dw2 = jnp.where(m2, w2, dw)
            r2 = jnp.where(m2, d, r2)
            w2 = jnp.where(m2, dw, w2)
            m3 = d2 > r3
            r3 = jnp.where(m3, d2, r3)
            w3 = jnp.where(m3, dw2, w3)
        r1_ref[sl, :] = r1
        r2_ref[sl, :] = r2
        r3_ref[sl, :] = r3
        w1_ref[sl, :] = w1
        w2_ref[sl, :] = w2
        w3_ref[sl, :] = w3


def _extract10(r_refs, p_refs, Bt):
    cand = jnp.concatenate([r[...] for r in r_refs], axis=1)
    wlc = jnp.concatenate([p[...] for p in p_refs], axis=1)
    lane = jax.lax.broadcasted_iota(jnp.int32, (Bt, 128), 1)
    newv = jnp.full((Bt, 128), -jnp.inf, jnp.float32)
    neww = jnp.zeros((Bt, 128), jnp.float32)
    for r in range(K):
        m = jnp.max(cand, axis=1, keepdims=True)
        eq = cand == m
        w = jnp.max(jnp.where(eq, wlc, -jnp.inf), axis=1, keepdims=True)
        cand = jnp.where(eq, -jnp.inf, cand)
        newv = jnp.where(lane == r, m, newv)
        neww = jnp.where(lane == r, w, neww)
    return newv, neww


def _init_state(r_refs, p_refs):
    for r in r_refs:
        r[...] = jnp.full_like(r, -jnp.inf)
    for p in p_refs:
        p[...] = jnp.zeros_like(p)


def _chunk_logits(qk_ref, ent_ref, c, n_valid):
    C = ent_ref.shape[0]
    ent = ent_ref[...]
    rn = jax.lax.rsqrt(jnp.sum(ent * ent, axis=1, keepdims=True))
    entn = ent * rn
    lo = jax.lax.dot_general(qk_ref[...], entn, (((1,), (1,)), ((), ())),
                             preferred_element_type=jnp.float32)
    col = jax.lax.broadcasted_iota(jnp.int32, (1, C), 1) + c * C
    return jnp.where(col < n_valid, lo, -jnp.inf), entn


def _retrieve_idx_body(n_valid, nchunks, qk_ref, ent_ref, attn_ref, idx_ref,
                       r1_ref, r2_ref, r3_ref, i1_ref, i2_ref, i3_ref):
    c = pl.program_id(1)
    C = ent_ref.shape[0]
    Bt = qk_ref.shape[0]
    r_refs = (r1_ref, r2_ref, r3_ref)
    p_refs = (i1_ref, i2_ref, i3_ref)

    @pl.when(c == 0)
    def _():
        _init_state(r_refs, p_refs)

    lo, _ = _chunk_logits(qk_ref, ent_ref, c, n_valid)
    lane_f = jax.lax.broadcasted_iota(jnp.int32, (32, 128), 1).astype(jnp.float32)
    base = (c * C).astype(jnp.float32)

    def partner_groups(g):
        return lane_f + (base + jnp.float32(g * 128))

    _insert3(lo, None, c, C, Bt, r_refs, p_refs, partner_groups)

    @pl.when(c == nchunks - 1)
    def _():
        newv, newi = _extract10(r_refs, p_refs, Bt)
        mx = jnp.max(newv, axis=1, keepdims=True)
        e = jnp.exp(newv - mx)
        s = jnp.sum(e, axis=1, keepdims=True)
        attn = e / s
        attn_ref[...] = attn[:, :16]
        idx_ref[...] = newi[:, :16].astype(jnp.int32)


def _retrieve_idx(ent, qk, C):
    n = ent.shape[0]
    Btot = qk.shape[0]
    Bt = 512
    nchunks = pl.cdiv(n, C)
    attn, idx = pl.pallas_call(
        functools.partial(_retrieve_idx_body, n, nchunks),
        grid=(Btot // Bt, nchunks),
        in_specs=[
            pl.BlockSpec((Bt, D), lambda b, c: (b, 0)),
            pl.BlockSpec((C, D), lambda b, c: (c, 0)),
        ],
        out_specs=[pl.BlockSpec((Bt, 16), lambda b, c: (b, 0))] * 2,
        out_shape=[jax.ShapeDtypeStruct((Btot, 16), jnp.float32),
                   jax.ShapeDtypeStruct((Btot, 16), jnp.int32)],
        scratch_shapes=[pltpu.VMEM((Bt, 128), jnp.float32)] * 6,
    )(qk, ent)
    return attn, idx


def _retrieve_wl_body(n_valid, nchunks, qk_ref, qv_ref, ent_ref, out_ref,
                      r1_ref, r2_ref, r3_ref, w1_ref, w2_ref, w3_ref):
    c = pl.program_id(1)
    C = ent_ref.shape[0]
    Bt = qk_ref.shape[0]
    r_refs = (r1_ref, r2_ref, r3_ref)
    p_refs = (w1_ref, w2_ref, w3_ref)

    @pl.when(c == 0)
    def _():
        _init_state(r_refs, p_refs)

    lo, entn = _chunk_logits(qk_ref, ent_ref, c, n_valid)
    wl = jax.lax.dot_general(qv_ref[...], entn, (((1,), (1,)), ((), ())),
                             preferred_element_type=jnp.float32)
    _insert3(lo, wl, c, C, Bt, r_refs, p_refs)

    @pl.when(c == nchunks - 1)
    def _():
        newv, neww = _extract10(r_refs, p_refs, Bt)
        mx = jnp.max(newv, axis=1, keepdims=True)
        e = jnp.exp(newv - mx)
        s = jnp.sum(e, axis=1, keepdims=True)
        score = jnp.sum((e / s) * neww, axis=1)
        out_ref[...] = score.reshape(out_ref.shape)


def _retrieve_wl(ent, qk, qv, C):
    n = ent.shape[0]
    Btot = qk.shape[0]
    Bt = 512
    nchunks = pl.cdiv(n, C)
    out = pl.pallas_call(
        functools.partial(_retrieve_wl_body, n, nchunks),
        grid=(Btot // Bt, nchunks),
        in_specs=[
            pl.BlockSpec((Bt, D), lambda b, c: (b, 0)),
            pl.BlockSpec((Bt, D), lambda b, c: (b, 0)),
            pl.BlockSpec((C, D), lambda b, c: (c, 0)),
        ],
        out_specs=pl.BlockSpec((1, Bt // 128, 128), lambda b, c: (b, 0, 0)),
        out_shape=jax.ShapeDtypeStruct((Btot // Bt, Bt // 128, 128), jnp.float32),
        scratch_shapes=[pltpu.VMEM((Bt, 128), jnp.float32)] * 6,
    )(qk, qv, ent)
    return out.reshape(Btot)


def _sc_combine(vals, idx, attn, qp):
    W, RPW = attn.shape[0], attn.shape[1]
    nb = (RPW * 16) // GB
    rpb = GB // 16
    mesh = plsc.VectorSubcoreMesh(core_axis_name="c", subcore_axis_name="s")

    @functools.partial(
        pl.kernel, mesh=mesh,
        out_type=jax.ShapeDtypeStruct((W, RPW, 16), jnp.float32),
        scratch_types=[
            pltpu.VMEM((nb, GB), jnp.int32),
            pltpu.VMEM((RPW, 16), jnp.float32),
            pltpu.VMEM((RPW, D), jnp.float32),
            pltpu.VMEM((GB, D), jnp.float32),
            pltpu.VMEM((RPW, 16), jnp.float32),
            pltpu.SemaphoreType.DMA,
        ],
    )
    def k(vals_hbm, idx_hbm, attn_hbm, qp_hbm, out_hbm,
          idx_v, attn_v, qp_v, rows_v, out_v, sem):
        wid = lax.axis_index("s") * 2 + lax.axis_index("c")
        pltpu.sync_copy(idx_hbm.at[wid], idx_v)
        pltpu.sync_copy(attn_hbm.at[wid], attn_v)
        pltpu.sync_copy(qp_hbm.at[wid], qp_v)
        for kb in range(nb):
            pltpu.async_copy(vals_hbm.at[idx_v.at[kb]], rows_v, sem).wait()

            def row_body(j, carry):
                r = kb * rpb + j
                av = attn_v[r, :]
                a = [av[t] for t in range(K)]
                sv = jnp.zeros((16,), jnp.float32)
                for d in range(D // 16):
                    ctx = a[0] * rows_v[j * 16, pl.ds(d * 16, 16)]
                    for t in range(1, K):
                        ctx = ctx + a[t] * rows_v[j * 16 + t, pl.ds(d * 16, 16)]
                    sv = sv + ctx * qp_v[r, pl.ds(d * 16, 16)]
                out_v[r, :] = sv
                return carry

            lax.fori_loop(0, rpb, row_body, 0)
        pltpu.sync_copy(out_v, out_hbm.at[wid])

    return k(vals, idx, attn, qp)


def _rowsum_body(in_ref, out_ref):
    out_ref[...] = jnp.sum(in_ref[...], axis=1).reshape(out_ref.shape)


def _rowsum(x):
    n = x.shape[0]
    Bt = 512
    out = pl.pallas_call(
        _rowsum_body,
        grid=(n // Bt,),
        in_specs=[pl.BlockSpec((Bt, 16), lambda b: (b, 0))],
        out_specs=pl.BlockSpec((1, Bt // 128, 128), lambda b: (b, 0, 0)),
        out_shape=jax.ShapeDtypeStruct((n // Bt, Bt // 128, 128), jnp.float32),
    )(x)
    return out.reshape(n)


def kernel(subj_q, rel_q, obj_q, entity_embeddings, relation_embeddings,
           Wq_subj, Wq_rel, Wq_obj, Wk_e, Wv_e, Wk_r, Wv_r, logit_scale):
    B = subj_q.shape[0]
    ls = jnp.reshape(logit_scale.astype(jnp.float32), (1,))
    qp_s, qk_s, _ = _prep(subj_q, Wq_subj, Wk_e, Wv_e, ls)
    qp_o, qk_o, _ = _prep(obj_q, Wq_obj, Wk_e, Wv_e, ls)
    _, qk_r, qv_r = _prep(rel_q, Wq_rel, Wk_r, Wv_r, ls)

    qk_so = jnp.concatenate([qk_s, qk_o], axis=0)
    qp_so = jnp.concatenate([qp_s, qp_o], axis=0)
    attn, idx = _retrieve_idx(entity_embeddings, qk_so, C=2048)
    vals = _vals_table(entity_embeddings, Wv_e)

    Btot = 2 * B
    rpw = Btot // NSC
    parts = _sc_combine(
        vals,
        idx.reshape(NSC, (rpw * 16) // GB, GB),
        attn.reshape(NSC, rpw, 16),
        qp_so.reshape(NSC, rpw, D),
    )
    so = _rowsum(parts.reshape(Btot, 16))

    r = _retrieve_wl(relation_embeddings, qk_r, qv_r, C=1024)
    return jnp.stack([so[:B], r, so[B:]], axis=0)


# speedup vs baseline: 1.0075x; 1.0075x over previous
import functools

import jax
import jax.numpy as jnp
from jax import lax
from jax.experimental import pallas as pl
from jax.experimental.pallas import tpu as pltpu
from jax.experimental.pallas import tpu_sc as plsc

D = 128
K = 10
NSC = 32
GB = 128


def _prep_body(ls_ref, q_ref, wq_ref, wk_ref, wv_ref, qp_ref, qk_ref, qv_ref):
    q = q_ref[...]
    rn = jax.lax.rsqrt(jnp.sum(q * q, axis=1, keepdims=True))
    qn = q * rn
    qp = jax.lax.dot_general(qn, wq_ref[...], (((1,), (1,)), ((), ())),
                             preferred_element_type=jnp.float32)
    scale = jnp.exp(ls_ref[0])
    qp_ref[...] = qp
    qk_ref[...] = scale * jax.lax.dot_general(qp, wk_ref[...], (((1,), (0,)), ((), ())),
                                              preferred_element_type=jnp.float32)
    qv_ref[...] = jax.lax.dot_general(qp, wv_ref[...], (((1,), (0,)), ((), ())),
                                      preferred_element_type=jnp.float32)


def _prep(q, wq, wk, wv, ls):
    B = q.shape[0]
    Bt = min(512, B)
    qp, qk, qv = pl.pallas_call(
        _prep_body,
        grid=(B // Bt,),
        in_specs=[
            pl.BlockSpec(memory_space=pltpu.SMEM),
            pl.BlockSpec((Bt, D), lambda b: (b, 0)),
            pl.BlockSpec((D, D), lambda b: (0, 0)),
            pl.BlockSpec((D, D), lambda b: (0, 0)),
            pl.BlockSpec((D, D), lambda b: (0, 0)),
        ],
        out_specs=[pl.BlockSpec((Bt, D), lambda b: (b, 0))] * 3,
        out_shape=[jax.ShapeDtypeStruct((B, D), jnp.float32)] * 3,
    )(ls, q, wq, wk, wv)
    return qp, qk, qv


def _vals_body(ent_ref, wv_ref, out_ref):
    ent = ent_ref[...]
    rn = jax.lax.rsqrt(jnp.sum(ent * ent, axis=1, keepdims=True))
    out_ref[...] = jax.lax.dot_general(ent * rn, wv_ref[...],
                                       (((1,), (1,)), ((), ())),
                                       preferred_element_type=jnp.float32)


def _vals_table(ent, wv, C=2048):
    n = ent.shape[0]
    return pl.pallas_call(
        _vals_body,
        grid=(pl.cdiv(n, C),),
        in_specs=[
            pl.BlockSpec((C, D), lambda c: (c, 0)),
            pl.BlockSpec((D, D), lambda c: (0, 0)),
        ],
        out_specs=pl.BlockSpec((C, D), lambda c: (c, 0)),
        out_shape=jax.ShapeDtypeStruct((n, D), jnp.float32),
    )(ent, wv)


def _insert3(lo, pr, c, C, Bt, r_refs, p_refs, partner_groups=None):
    r1_ref, r2_ref, r3_ref = r_refs
    w1_ref, w2_ref, w3_ref = p_refs
    RB = 32
    for i in range(Bt // RB):
        sl = pl.ds(i * RB, RB)
        r1 = r1_ref[sl, :]
        r2 = r2_ref[sl, :]
        r3 = r3_ref[sl, :]
        w1 = w1_ref[sl, :]
        w2 = w2_ref[sl, :]
        w3 = w3_ref[sl, :]
        for g in range(C // 128):
            x = lo[i * RB:(i + 1) * RB, g * 128:(g + 1) * 128]
            if pr is not None:
                wx = pr[i * RB:(i + 1) * RB, g * 128:(g + 1) * 128]
            else:
                wx = partner_groups(g)
            m = x > r1
            d = jnp.where(m, r1, x)
            dw = jnp.where(m, w1, wx)
            r1 = jnp.where(m, x, r1)
            w1 = jnp.where(m, wx, w1)
            m2 = d > r2
            d2 = jnp.where(m2, r2, d)
            dw2 = jnp.where(m2, w2, dw)
            r2 = jnp.where(m2, d, r2)
            w2 = jnp.where(m2, dw, w2)
            m3 = d2 > r3
            r3 = jnp.where(m3, d2, r3)
            w3 = jnp.where(m3, dw2, w3)
        r1_ref[sl, :] = r1
        r2_ref[sl, :] = r2
        r3_ref[sl, :] = r3
        w1_ref[sl, :] = w1
        w2_ref[sl, :] = w2
        w3_ref[sl, :] = w3


def _extract10(r_refs, p_refs, Bt):
    cand = jnp.concatenate([r[...] for r in r_refs], axis=1)
    wlc = jnp.concatenate([p[...] for p in p_refs], axis=1)
    lane = jax.lax.broadcasted_iota(jnp.int32, (Bt, 128), 1)
    newv = jnp.full((Bt, 128), -jnp.inf, jnp.float32)
    neww = jnp.zeros((Bt, 128), jnp.float32)
    for r in range(K):
        m = jnp.max(cand, axis=1, keepdims=True)
        eq = cand == m
        w = jnp.max(jnp.where(eq, wlc, -jnp.inf), axis=1, keepdims=True)
        cand = jnp.where(eq, -jnp.inf, cand)
        newv = jnp.where(lane == r, m, newv)
        neww = jnp.where(lane == r, w, neww)
    return newv, neww


def _init_state(r_refs, p_refs):
    for r in r_refs:
        r[...] = jnp.full_like(r, -jnp.inf)
    for p in p_refs:
        p[...] = jnp.zeros_like(p)


def _chunk_logits(qk_ref, ent_ref, c, n_valid):
    C = ent_ref.shape[0]
    ent = ent_ref[...]
    rn = jax.lax.rsqrt(jnp.sum(ent * ent, axis=1, keepdims=True))
    entn = ent * rn
    lo = jax.lax.dot_general(qk_ref[...], entn, (((1,), (1,)), ((), ())),
                             preferred_element_type=jnp.float32)
    col = jax.lax.broadcasted_iota(jnp.int32, (1, C), 1) + c * C
    return jnp.where(col < n_valid, lo, -jnp.inf), entn


def _retrieve_idx_body(n_valid, nchunks, qk_ref, ent_ref, attn_ref, idx_ref,
                       r1_ref, r2_ref, r3_ref, i1_ref, i2_ref, i3_ref):
    c = pl.program_id(1)
    C = ent_ref.shape[0]
    Bt = qk_ref.shape[0]
    r_refs = (r1_ref, r2_ref, r3_ref)
    p_refs = (i1_ref, i2_ref, i3_ref)

    @pl.when(c == 0)
    def _():
        _init_state(r_refs, p_refs)

    lo, _ = _chunk_logits(qk_ref, ent_ref, c, n_valid)
    lane_f = jax.lax.broadcasted_iota(jnp.int32, (32, 128), 1).astype(jnp.float32)
    base = (c * C).astype(jnp.float32)

    def partner_groups(g):
        return lane_f + (base + jnp.float32(g * 128))

    _insert3(lo, None, c, C, Bt, r_refs, p_refs, partner_groups)

    @pl.when(c == nchunks - 1)
    def _():
        newv, newi = _extract10(r_refs, p_refs, Bt)
        mx = jnp.max(newv, axis=1, keepdims=True)
        e = jnp.exp(newv - mx)
        s = jnp.sum(e, axis=1, keepdims=True)
        attn = e / s
        attn_ref[...] = attn[:, :16]
        idx_ref[...] = newi[:, :16].astype(jnp.int32)


def _retrieve_idx(ent, qk, C):
    n = ent.shape[0]
    Btot = qk.shape[0]
    Bt = 512
    nchunks = pl.cdiv(n, C)
    attn, idx = pl.pallas_call(
        functools.partial(_retrieve_idx_body, n, nchunks),
        grid=(Btot // Bt, nchunks),
        in_specs=[
            pl.BlockSpec((Bt, D), lambda b, c: (b, 0)),
            pl.BlockSpec((C, D), lambda b, c: (c, 0)),
        ],
        out_specs=[pl.BlockSpec((Bt, 16), lambda b, c: (b, 0))] * 2,
        out_shape=[jax.ShapeDtypeStruct((Btot, 16), jnp.float32),
                   jax.ShapeDtypeStruct((Btot, 16), jnp.int32)],
        scratch_shapes=[pltpu.VMEM((Bt, 128), jnp.float32)] * 6,
    )(qk, ent)
    return attn, idx


def _retrieve_wl_body(n_valid, nchunks, qk_ref, qv_ref, ent_ref, out_ref,
                      r1_ref, r2_ref, r3_ref, w1_ref, w2_ref, w3_ref):
    c = pl.program_id(1)
    C = ent_ref.shape[0]
    Bt = qk_ref.shape[0]
    r_refs = (r1_ref, r2_ref, r3_ref)
    p_refs = (w1_ref, w2_ref, w3_ref)

    @pl.when(c == 0)
    def _():
        _init_state(r_refs, p_refs)

    lo, entn = _chunk_logits(qk_ref, ent_ref, c, n_valid)
    wl = jax.lax.dot_general(qv_ref[...], entn, (((1,), (1,)), ((), ())),
                             preferred_element_type=jnp.float32)
    _insert3(lo, wl, c, C, Bt, r_refs, p_refs)

    @pl.when(c == nchunks - 1)
    def _():
        newv, neww = _extract10(r_refs, p_refs, Bt)
        mx = jnp.max(newv, axis=1, keepdims=True)
        e = jnp.exp(newv - mx)
        s = jnp.sum(e, axis=1, keepdims=True)
        score = jnp.sum((e / s) * neww, axis=1)
        out_ref[...] = score.reshape(out_ref.shape)


def _retrieve_wl(ent, qk, qv, C):
    n = ent.shape[0]
    Btot = qk.shape[0]
    Bt = 512
    nchunks = pl.cdiv(n, C)
    out = pl.pallas_call(
        functools.partial(_retrieve_wl_body, n, nchunks),
        grid=(Btot // Bt, nchunks),
        in_specs=[
            pl.BlockSpec((Bt, D), lambda b, c: (b, 0)),
            pl.BlockSpec((Bt, D), lambda b, c: (b, 0)),
            pl.BlockSpec((C, D), lambda b, c: (c, 0)),
        ],
        out_specs=pl.BlockSpec((1, Bt // 128, 128), lambda b, c: (b, 0, 0)),
        out_shape=jax.ShapeDtypeStruct((Btot // Bt, Bt // 128, 128), jnp.float32),
        scratch_shapes=[pltpu.VMEM((Bt, 128), jnp.float32)] * 6,
    )(qk, qv, ent)
    return out.reshape(Btot)


def _sc_combine(vals, idx, attn, qp):
    W, RPW = attn.shape[0], attn.shape[1]
    nb = (RPW * 16) // GB
    rpb = GB // 16
    mesh = plsc.VectorSubcoreMesh(core_axis_name="c", subcore_axis_name="s")

    @functools.partial(
        pl.kernel, mesh=mesh,
        out_type=jax.ShapeDtypeStruct((W, RPW, 16), jnp.float32),
        scratch_types=[
            pltpu.VMEM((nb, GB), jnp.int32),
            pltpu.VMEM((RPW, 16), jnp.float32),
            pltpu.VMEM((RPW, D), jnp.float32),
            pltpu.VMEM((GB, D), jnp.float32),
            pltpu.VMEM((RPW, 16), jnp.float32),
            pltpu.SemaphoreType.DMA,
        ],
    )
    def k(vals_hbm, idx_hbm, attn_hbm, qp_hbm, out_hbm,
          idx_v, attn_v, qp_v, rows_v, out_v, sem):
        wid = lax.axis_index("s") * 2 + lax.axis_index("c")
        pltpu.sync_copy(idx_hbm.at[wid], idx_v)
        pltpu.sync_copy(attn_hbm.at[wid], attn_v)
        pltpu.sync_copy(qp_hbm.at[wid], qp_v)
        def batch_body(kb, carry):
            pltpu.async_copy(vals_hbm.at[idx_v.at[kb]], rows_v, sem).wait()
            for j in range(rpb):
                r = kb * rpb + j
                av = attn_v[r, :]
                a = [av[t] for t in range(K)]
                sv = jnp.zeros((16,), jnp.float32)
                for d in range(D // 16):
                    ctx = a[0] * rows_v[j * 16, pl.ds(d * 16, 16)]
                    for t in range(1, K):
                        ctx = ctx + a[t] * rows_v[j * 16 + t, pl.ds(d * 16, 16)]
                    sv = sv + ctx * qp_v[r, pl.ds(d * 16, 16)]
                out_v[r, :] = sv
            return carry

        lax.fori_loop(0, nb, batch_body, 0)
        pltpu.sync_copy(out_v, out_hbm.at[wid])

    return k(vals, idx, attn, qp)


def _rowsum_body(in_ref, out_ref):
    out_ref[...] = jnp.sum(in_ref[...], axis=1).reshape(out_ref.shape)


def _rowsum(x):
    n = x.shape[0]
    Bt = 512
    out = pl.pallas_call(
        _rowsum_body,
        grid=(n // Bt,),
        in_specs=[pl.BlockSpec((Bt, 16), lambda b: (b, 0))],
        out_specs=pl.BlockSpec((1, Bt // 128, 128), lambda b: (b, 0, 0)),
        out_shape=jax.ShapeDtypeStruct((n // Bt, Bt // 128, 128), jnp.float32),
    )(x)
    return out.reshape(n)


def kernel(subj_q, rel_q, obj_q, entity_embeddings, relation_embeddings,
           Wq_subj, Wq_rel, Wq_obj, Wk_e, Wv_e, Wk_r, Wv_r, logit_scale):
    B = subj_q.shape[0]
    ls = jnp.reshape(logit_scale.astype(jnp.float32), (1,))
    qp_s, qk_s, _ = _prep(subj_q, Wq_subj, Wk_e, Wv_e, ls)
    qp_o, qk_o, _ = _prep(obj_q, Wq_obj, Wk_e, Wv_e, ls)
    _, qk_r, qv_r = _prep(rel_q, Wq_rel, Wk_r, Wv_r, ls)

    qk_so = jnp.concatenate([qk_s, qk_o], axis=0)
    qp_so = jnp.concatenate([qp_s, qp_o], axis=0)
    attn, idx = _retrieve_idx(entity_embeddings, qk_so, C=2048)
    vals = _vals_table(entity_embeddings, Wv_e)

    Btot = 2 * B
    rpw = Btot // NSC
    parts = _sc_combine(
        vals,
        idx.reshape(NSC, (rpw * 16) // GB, GB),
        attn.reshape(NSC, rpw, 16),
        qp_so.reshape(NSC, rpw, D),
    )
    so = _rowsum(parts.reshape(Btot, 16))

    r = _retrieve_wl(relation_embeddings, qk_r, qv_r, C=1024)
    return jnp.stack([so[:B], r, so[B:]], axis=0)


# speedup vs baseline: 1.0086x; 1.0011x over previous
import functools

import jax
import jax.numpy as jnp
from jax import lax
from jax.experimental import pallas as pl
from jax.experimental.pallas import tpu as pltpu
from jax.experimental.pallas import tpu_sc as plsc

D = 128
K = 10
NSC = 32
GB = 128


def _prep_body(ls_ref, q_ref, wq_ref, wk_ref, wv_ref, qp_ref, qk_ref, qv_ref):
    q = q_ref[...]
    rn = jax.lax.rsqrt(jnp.sum(q * q, axis=1, keepdims=True))
    qn = q * rn
    qp = jax.lax.dot_general(qn, wq_ref[...], (((1,), (1,)), ((), ())),
                             preferred_element_type=jnp.float32)
    scale = jnp.exp(ls_ref[0])
    qp_ref[...] = qp
    qk_ref[...] = scale * jax.lax.dot_general(qp, wk_ref[...], (((1,), (0,)), ((), ())),
                                              preferred_element_type=jnp.float32)
    qv_ref[...] = jax.lax.dot_general(qp, wv_ref[...], (((1,), (0,)), ((), ())),
                                      preferred_element_type=jnp.float32)


def _prep(q, wq, wk, wv, ls):
    B = q.shape[0]
    Bt = min(512, B)
    qp, qk, qv = pl.pallas_call(
        _prep_body,
        grid=(B // Bt,),
        in_specs=[
            pl.BlockSpec(memory_space=pltpu.SMEM),
            pl.BlockSpec((Bt, D), lambda b: (b, 0)),
            pl.BlockSpec((D, D), lambda b: (0, 0)),
            pl.BlockSpec((D, D), lambda b: (0, 0)),
            pl.BlockSpec((D, D), lambda b: (0, 0)),
        ],
        out_specs=[pl.BlockSpec((Bt, D), lambda b: (b, 0))] * 3,
        out_shape=[jax.ShapeDtypeStruct((B, D), jnp.float32)] * 3,
    )(ls, q, wq, wk, wv)
    return qp, qk, qv


def _vals_body(ent_ref, wv_ref, out_ref):
    ent = ent_ref[...]
    rn = jax.lax.rsqrt(jnp.sum(ent * ent, axis=1, keepdims=True))
    out_ref[...] = jax.lax.dot_general(ent * rn, wv_ref[...],
                                       (((1,), (1,)), ((), ())),
                                       preferred_element_type=jnp.float32)


def _vals_table(ent, wv, C=2048):
    n = ent.shape[0]
    return pl.pallas_call(
        _vals_body,
        grid=(pl.cdiv(n, C),),
        in_specs=[
            pl.BlockSpec((C, D), lambda c: (c, 0)),
            pl.BlockSpec((D, D), lambda c: (0, 0)),
        ],
        out_specs=pl.BlockSpec((C, D), lambda c: (c, 0)),
        out_shape=jax.ShapeDtypeStruct((n, D), jnp.float32),
    )(ent, wv)


def _insert3(lo, pr, c, C, Bt, r_refs, p_refs, partner_groups=None):
    r1_ref, r2_ref, r3_ref = r_refs
    w1_ref, w2_ref, w3_ref = p_refs
    RB = 32
    for i in range(Bt // RB):
        sl = pl.ds(i * RB, RB)
        r1 = r1_ref[sl, :]
        r2 = r2_ref[sl, :]
        r3 = r3_ref[sl, :]
        w1 = w1_ref[sl, :]
        w2 = w2_ref[sl, :]
        w3 = w3_ref[sl, :]
        for g in range(C // 128):
            x = lo[i * RB:(i + 1) * RB, g * 128:(g + 1) * 128]
            if pr is not None:
                wx = pr[i * RB:(i + 1) * RB, g * 128:(g + 1) * 128]
            else:
                wx = partner_groups(g)
            m = x > r1
            d = jnp.where(m, r1, x)
            dw = jnp.where(m, w1, wx)
            r1 = jnp.where(m, x, r1)
            w1 = jnp.where(m, wx, w1)
            m2 = d > r2
            d2 = jnp.where(m2, r2, d)
            dw2 = jnp.where(m2, w2, dw)
            r2 = jnp.where(m2, d, r2)
            w2 = jnp.where(m2, dw, w2)
            m3 = d2 > r3
            r3 = jnp.where(m3, d2, r3)
            w3 = jnp.where(m3, dw2, w3)
        r1_ref[sl, :] = r1
        r2_ref[sl, :] = r2
        r3_ref[sl, :] = r3
        w1_ref[sl, :] = w1
        w2_ref[sl, :] = w2
        w3_ref[sl, :] = w3


def _extract10(r_refs, p_refs, Bt):
    cand = jnp.concatenate([r[...] for r in r_refs], axis=1)
    wlc = jnp.concatenate([p[...] for p in p_refs], axis=1)
    lane = jax.lax.broadcasted_iota(jnp.int32, (Bt, 128), 1)
    newv = jnp.full((Bt, 128), -jnp.inf, jnp.float32)
    neww = jnp.zeros((Bt, 128), jnp.float32)
    for r in range(K):
        m = jnp.max(cand, axis=1, keepdims=True)
        eq = cand == m
        w = jnp.max(jnp.where(eq, wlc, -jnp.inf), axis=1, keepdims=True)
        cand = jnp.where(eq, -jnp.inf, cand)
        newv = jnp.where(lane == r, m, newv)
        neww = jnp.where(lane == r, w, neww)
    return newv, neww


def _init_state(r_refs, p_refs):
    for r in r_refs:
        r[...] = jnp.full_like(r, -jnp.inf)
    for p in p_refs:
        p[...] = jnp.zeros_like(p)


def _chunk_logits(qk_ref, ent_ref, c, n_valid):
    C = ent_ref.shape[0]
    ent = ent_ref[...]
    rn = jax.lax.rsqrt(jnp.sum(ent * ent, axis=1, keepdims=True))
    entn = ent * rn
    lo = jax.lax.dot_general(qk_ref[...], entn, (((1,), (1,)), ((), ())),
                             preferred_element_type=jnp.float32)
    col = jax.lax.broadcasted_iota(jnp.int32, (1, C), 1) + c * C
    return jnp.where(col < n_valid, lo, -jnp.inf), entn


def _retrieve_idx_body(n_valid, nchunks, qk_ref, ent_ref, attn_ref, idx_ref,
                       r1_ref, r2_ref, r3_ref, i1_ref, i2_ref, i3_ref):
    c = pl.program_id(1)
    C = ent_ref.shape[0]
    Bt = qk_ref.shape[0]
    r_refs = (r1_ref, r2_ref, r3_ref)
    p_refs = (i1_ref, i2_ref, i3_ref)

    @pl.when(c == 0)
    def _():
        _init_state(r_refs, p_refs)

    lo, _ = _chunk_logits(qk_ref, ent_ref, c, n_valid)
    lane_f = jax.lax.broadcasted_iota(jnp.int32, (32, 128), 1).astype(jnp.float32)
    base = (c * C).astype(jnp.float32)

    def partner_groups(g):
        return lane_f + (base + jnp.float32(g * 128))

    _insert3(lo, None, c, C, Bt, r_refs, p_refs, partner_groups)

    @pl.when(c == nchunks - 1)
    def _():
        newv, newi = _extract10(r_refs, p_refs, Bt)
        mx = jnp.max(newv, axis=1, keepdims=True)
        e = jnp.exp(newv - mx)
        s = jnp.sum(e, axis=1, keepdims=True)
        attn = e / s
        attn_ref[...] = attn[:, :16]
        idx_ref[...] = newi[:, :16].astype(jnp.int32)


def _retrieve_idx(ent, qk, C):
    n = ent.shape[0]
    Btot = qk.shape[0]
    Bt = 512
    nchunks = pl.cdiv(n, C)
    attn, idx = pl.pallas_call(
        functools.partial(_retrieve_idx_body, n, nchunks),
        grid=(Btot // Bt, nchunks),
        in_specs=[
            pl.BlockSpec((Bt, D), lambda b, c: (b, 0)),
            pl.BlockSpec((C, D), lambda b, c: (c, 0)),
        ],
        out_specs=[pl.BlockSpec((Bt, 16), lambda b, c: (b, 0))] * 2,
        out_shape=[jax.ShapeDtypeStruct((Btot, 16), jnp.float32),
                   jax.ShapeDtypeStruct((Btot, 16), jnp.int32)],
        scratch_shapes=[pltpu.VMEM((Bt, 128), jnp.float32)] * 6,
    )(qk, ent)
    return attn, idx


def _retrieve_wl_body(n_valid, nchunks, qk_ref, qv_ref, ent_ref, out_ref,
                      r1_ref, r2_ref, r3_ref, w1_ref, w2_ref, w3_ref):
    c = pl.program_id(1)
    C = ent_ref.shape[0]
    Bt = qk_ref.shape[0]
    r_refs = (r1_ref, r2_ref, r3_ref)
    p_refs = (w1_ref, w2_ref, w3_ref)

    @pl.when(c == 0)
    def _():
        _init_state(r_refs, p_refs)

    lo, entn = _chunk_logits(qk_ref, ent_ref, c, n_valid)
    wl = jax.lax.dot_general(qv_ref[...], entn, (((1,), (1,)), ((), ())),
                             preferred_element_type=jnp.float32)
    _insert3(lo, wl, c, C, Bt, r_refs, p_refs)

    @pl.when(c == nchunks - 1)
    def _():
        newv, neww = _extract10(r_refs, p_refs, Bt)
        mx = jnp.max(newv, axis=1, keepdims=True)
        e = jnp.exp(newv - mx)
        s = jnp.sum(e, axis=1, keepdims=True)
        score = jnp.sum((e / s) * neww, axis=1)
        out_ref[...] = score.reshape(out_ref.shape)


def _retrieve_wl(ent, qk, qv, C):
    n = ent.shape[0]
    Btot = qk.shape[0]
    Bt = 512
    nchunks = pl.cdiv(n, C)
    out = pl.pallas_call(
        functools.partial(_retrieve_wl_body, n, nchunks),
        grid=(Btot // Bt, nchunks),
        in_specs=[
            pl.BlockSpec((Bt, D), lambda b, c: (b, 0)),
            pl.BlockSpec((Bt, D), lambda b, c: (b, 0)),
            pl.BlockSpec((C, D), lambda b, c: (c, 0)),
        ],
        out_specs=pl.BlockSpec((1, Bt // 128, 128), lambda b, c: (b, 0, 0)),
        out_shape=jax.ShapeDtypeStruct((Btot // Bt, Bt // 128, 128), jnp.float32),
        scratch_shapes=[pltpu.VMEM((Bt, 128), jnp.float32)] * 6,
    )(qk, qv, ent)
    return out.reshape(Btot)


def _sc_combine(vals, idx, attn, qp):
    W, RPW = attn.shape[0], attn.shape[1]
    nb = (RPW * 16) // GB
    rpb = GB // 16
    mesh = plsc.VectorSubcoreMesh(core_axis_name="c", subcore_axis_name="s")

    @functools.partial(
        pl.kernel, mesh=mesh,
        out_type=jax.ShapeDtypeStruct((W, RPW, 16), jnp.float32),
        scratch_types=[
            pltpu.VMEM((nb, GB), jnp.int32),
            pltpu.VMEM((RPW, 16), jnp.float32),
            pltpu.VMEM((16, D), jnp.float32),
            pltpu.VMEM((2 * GB, D), jnp.float32),
            pltpu.VMEM((RPW, 16), jnp.float32),
            pltpu.SemaphoreType.DMA,
        ],
    )
    def k(vals_hbm, idx_hbm, attn_hbm, qp_hbm, out_hbm,
          idx_v, attn_v, qp_v, rows_v, out_v, sem):
        wid = lax.axis_index("s") * 2 + lax.axis_index("c")
        pltpu.sync_copy(idx_hbm.at[wid], idx_v)
        pltpu.sync_copy(attn_hbm.at[wid], attn_v)
        NF = 2

        def macro_body(mb, carry):
            handles = []
            for q in range(NF):
                kb = mb * NF + q
                handles.append(pltpu.async_copy(
                    vals_hbm.at[idx_v.at[kb]],
                    rows_v.at[pl.ds(q * GB, GB)], sem))
            pltpu.sync_copy(qp_hbm.at[wid, pl.ds(mb * NF * rpb, NF * rpb)],
                            qp_v)
            for q in range(NF):
                handles[q].wait()
                kb = mb * NF + q
                for j in range(rpb):
                    r = kb * rpb + j
                    lr = q * rpb + j
                    jj = q * GB + j * 16
                    av = attn_v[r, :]
                    a = [av[t] for t in range(K)]
                    sv = jnp.zeros((16,), jnp.float32)
                    for d in range(D // 16):
                        ctx = a[0] * rows_v[jj, pl.ds(d * 16, 16)]
                        for t in range(1, K):
                            ctx = ctx + a[t] * rows_v[jj + t, pl.ds(d * 16, 16)]
                        sv = sv + ctx * qp_v[lr, pl.ds(d * 16, 16)]
                    out_v[r, :] = sv
            return carry

        lax.fori_loop(0, nb // NF, macro_body, 0)
        pltpu.sync_copy(out_v, out_hbm.at[wid])

    return k(vals, idx, attn, qp)


def _rowsum_body(in_ref, out_ref):
    out_ref[...] = jnp.sum(in_ref[...], axis=1).reshape(out_ref.shape)


def _rowsum(x):
    n = x.shape[0]
    Bt = 512
    out = pl.pallas_call(
        _rowsum_body,
        grid=(n // Bt,),
        in_specs=[pl.BlockSpec((Bt, 16), lambda b: (b, 0))],
        out_specs=pl.BlockSpec((1, Bt // 128, 128), lambda b: (b, 0, 0)),
        out_shape=jax.ShapeDtypeStruct((n // Bt, Bt // 128, 128), jnp.float32),
    )(x)
    return out.reshape(n)


def kernel(subj_q, rel_q, obj_q, entity_embeddings, relation_embeddings,
           Wq_subj, Wq_rel, Wq_obj, Wk_e, Wv_e, Wk_r, Wv_r, logit_scale):
    B = subj_q.shape[0]
    ls = jnp.reshape(logit_scale.astype(jnp.float32), (1,))
    qp_s, qk_s, _ = _prep(subj_q, Wq_subj, Wk_e, Wv_e, ls)
    qp_o, qk_o, _ = _prep(obj_q, Wq_obj, Wk_e, Wv_e, ls)
    _, qk_r, qv_r = _prep(rel_q, Wq_rel, Wk_r, Wv_r, ls)

    qk_so = jnp.concatenate([qk_s, qk_o], axis=0)
    qp_so = jnp.concatenate([qp_s, qp_o], axis=0)
    attn, idx = _retrieve_idx(entity_embeddings, qk_so, C=2048)
    vals = _vals_table(entity_embeddings, Wv_e)

    Btot = 2 * B
    rpw = Btot // NSC
    parts = _sc_combine(
        vals,
        idx.reshape(NSC, (rpw * 16) // GB, GB),
        attn.reshape(NSC, rpw, 16),
        qp_so.reshape(NSC, rpw, D),
    )
    so = _rowsum(parts.reshape(Btot, 16))

    r = _retrieve_wl(relation_embeddings, qk_r, qv_r, C=1024)
    return jnp.stack([so[:B], r, so[B:]], axis=0)
